# Initial kernel scaffold; baseline (speedup 1.0000x reference)
import jax
import jax.numpy as jnp
from jax.experimental import pallas as pl


def kernel(grid, output_scaling_factors):
    raise NotImplementedError("write your pallas kernel here")


# speedup vs baseline: 6.3023x; 6.3023x over previous
import jax
import jax.numpy as jnp
from jax import lax
from jax.experimental import pallas as pl
from jax.experimental.pallas import tpu as pltpu
from jax.experimental.pallas import tpu_sc as plsc

SX = SY = SZ = 64
ROW = SY * SX
B = 16
N = SX * SY * SZ
PAD = 64
NC, NS, L = 2, 16, 16
NW = NC * NS


def _gridding_reverse_sc():
    mesh = plsc.VectorSubcoreMesh(
        core_axis_name="c", subcore_axis_name="s", num_cores=NC, num_subcores=NS
    )

    @pl.kernel(
        out_type=jax.ShapeDtypeStruct((B, SZ, 3 * ROW), jnp.float32),
        mesh=mesh,
        compiler_params=pltpu.CompilerParams(
            needs_layout_passes=False, use_tc_tiling_on_sc=False
        ),
        scratch_types=[
            pltpu.VMEM((PAD + ROW,), jnp.float32),
            pltpu.VMEM((PAD + ROW,), jnp.float32),
            pltpu.VMEM((PAD + ROW,), jnp.float32),
            pltpu.VMEM((PAD + ROW,), jnp.float32),
            pltpu.VMEM((3 * ROW,), jnp.float32),
            pltpu.VMEM((3 * L,), jnp.float32),
        ],
    )
    def k(grid_hbm, osf_hbm, out_hbm, gP, gC, sxP, sxC, out_v, osf_v):
        wid = lax.axis_index("s") * NC + lax.axis_index("c")
        b = wid >> 1
        z0 = (wid & 1) * (SZ // 2)

        pltpu.sync_copy(osf_hbm, osf_v)
        s0 = osf_v[pl.ds(0, L)]
        s1 = osf_v[pl.ds(L, L)]
        s2 = osf_v[pl.ds(2 * L, L)]

        iota = lax.iota(jnp.int32, L)
        idx3 = iota * 3
        zeros = jnp.zeros((L,), jnp.float32)

        def zbody(i, _):
            out_v[pl.ds(i * L, L)] = zeros
            return 0
        lax.fori_loop(0, 3 * ROW // L, zbody, 0)

        def pass1(gbuf, sxbuf):
            def body(i, _):
                c = i * L
                g0 = gbuf[pl.ds(PAD + c, L)]
                gm = plsc.load_gather(gbuf, [iota + (PAD - 1 + c)])
                sxbuf[pl.ds(PAD + c, L)] = g0 + gm
                return 0
            lax.fori_loop(0, ROW // L, body, 0)

        def slab(i, _):
            z = z0 + i

            @pl.when(z == 0)
            def _():
                pltpu.sync_copy(out_v, out_hbm.at[b, 0])

            @pl.when(z > 0)
            def _():
                pltpu.sync_copy(grid_hbm.at[b, z - 1], gP.at[pl.ds(PAD, ROW)])
                pltpu.sync_copy(grid_hbm.at[b, z], gC.at[pl.ds(PAD, ROW)])
                pass1(gP, sxP)
                pass1(gC, sxC)
                zf = z.astype(jnp.float32) - 32.0

                def body(v, _):
                    c = v * L
                    sxC0 = sxC[pl.ds(PAD + c, L)]
                    sxC1 = sxC[pl.ds(c, L)]
                    sxP0 = sxP[pl.ds(PAD + c, L)]
                    sxP1 = sxP[pl.ds(c, L)]
                    gC0 = gC[pl.ds(PAD + c, L)]
                    gC1 = gC[pl.ds(c, L)]
                    gP0 = gP[pl.ds(PAD + c, L)]
                    gP1 = gP[pl.ds(c, L)]
                    sy1 = sxC1 + sxP1
                    sz1 = sxP0 + sxP1
                    wsum = sxC0 + sxC1 + sz1
                    gsum = (gC0 + gC1) + (gP0 + gP1)
                    sx1 = wsum - gsum
                    r = 1.0 / wsum
                    xi = iota + (c & (SX - 1))
                    y = c >> 6
                    yf = y.astype(jnp.float32) - 32.0
                    xf = xi.astype(jnp.float32) - 32.0
                    px = (xf - sx1 * r) * s0
                    py = (yf - sy1 * r) * s1
                    pz = (zf - sz1 * r) * s2
                    m = (wsum != 0.0) & (xi > 0) & (y > 0)
                    base = idx3 + 3 * c
                    plsc.store_scatter(out_v, [base], jnp.where(m, px, 0.0))
                    plsc.store_scatter(out_v, [base + 1], jnp.where(m, py, 0.0))
                    plsc.store_scatter(out_v, [base + 2], jnp.where(m, pz, 0.0))
                    return 0

                lax.fori_loop(0, ROW // L, body, 0)
                pltpu.sync_copy(out_v, out_hbm.at[b, z])

            return 0

        lax.fori_loop(0, SZ // 2, slab, 0)

    return k


def kernel(grid, output_scaling_factors):
    osf_exp = jnp.repeat(output_scaling_factors, L)
    grid3 = grid.reshape(B, SZ, ROW)
    out_flat = _gridding_reverse_sc()(grid3, osf_exp)
    return out_flat.reshape(B, N, 3)


# speedup vs baseline: 24.4616x; 3.8814x over previous
import jax
import jax.numpy as jnp
from jax import lax
from jax.experimental import pallas as pl
from jax.experimental.pallas import tpu as pltpu
from jax.experimental.pallas import tpu_sc as plsc

SX = SY = SZ = 64
ROW = SY * SX
B = 16
N = SX * SY * SZ
PAD = 64
NC, NS, L = 2, 16, 16


def _gridding_reverse_sc():
    mesh = plsc.VectorSubcoreMesh(
        core_axis_name="c", subcore_axis_name="s", num_cores=NC, num_subcores=NS
    )

    @pl.kernel(
        out_type=jax.ShapeDtypeStruct((3, B, SZ, ROW), jnp.float32),
        mesh=mesh,
        compiler_params=pltpu.CompilerParams(
            needs_layout_passes=False, use_tc_tiling_on_sc=False
        ),
        scratch_types=[
            pltpu.VMEM((PAD + ROW,), jnp.float32),
            pltpu.VMEM((PAD + ROW,), jnp.float32),
            pltpu.VMEM((PAD + ROW,), jnp.float32),
            pltpu.VMEM((PAD + ROW,), jnp.float32),
            pltpu.VMEM((ROW,), jnp.float32),
            pltpu.VMEM((ROW,), jnp.float32),
            pltpu.VMEM((ROW,), jnp.float32),
            pltpu.VMEM((3 * L,), jnp.float32),
        ],
    )
    def k(grid_hbm, osf_hbm, out_hbm, gP, gC, sxP, sxC, pxv, pyv, pzv, osf_v):
        wid = lax.axis_index("s") * NC + lax.axis_index("c")
        b = wid >> 1
        z0 = (wid & 1) * (SZ // 2)

        pltpu.sync_copy(osf_hbm, osf_v)
        s0 = osf_v[pl.ds(0, L)]
        s1 = osf_v[pl.ds(L, L)]
        s2 = osf_v[pl.ds(2 * L, L)]

        iota = lax.iota(jnp.int32, L)
        zeros = jnp.zeros((L,), jnp.float32)

        def zbody(i, _):
            pxv[pl.ds(i * L, L)] = zeros
            pyv[pl.ds(i * L, L)] = zeros
            pzv[pl.ds(i * L, L)] = zeros
            return 0
        lax.fori_loop(0, ROW // L, zbody, 0)

        def pass1(gbuf, sxbuf):
            def body(i, _):
                c = i * L
                g0 = gbuf[pl.ds(PAD + c, L)]
                gm = plsc.load_gather(gbuf, [iota + (PAD - 1 + c)])
                sxbuf[pl.ds(PAD + c, L)] = g0 + gm
                return 0
            lax.fori_loop(0, ROW // L, body, 0)

        def slab(i, _):
            z = z0 + i

            @pl.when(z == 0)
            def _():
                pltpu.sync_copy(pxv, out_hbm.at[0, b, 0])
                pltpu.sync_copy(pyv, out_hbm.at[1, b, 0])
                pltpu.sync_copy(pzv, out_hbm.at[2, b, 0])

            @pl.when(z > 0)
            def _():
                pltpu.sync_copy(grid_hbm.at[b, z - 1], gP.at[pl.ds(PAD, ROW)])
                pltpu.sync_copy(grid_hbm.at[b, z], gC.at[pl.ds(PAD, ROW)])
                pass1(gP, sxP)
                pass1(gC, sxC)
                zf = z.astype(jnp.float32) - 32.0

                def body(v, _):
                    c = v * L
                    sxC0 = sxC[pl.ds(PAD + c, L)]
                    sxC1 = sxC[pl.ds(c, L)]
                    sxP0 = sxP[pl.ds(PAD + c, L)]
                    sxP1 = sxP[pl.ds(c, L)]
                    gC0 = gC[pl.ds(PAD + c, L)]
                    gC1 = gC[pl.ds(c, L)]
                    gP0 = gP[pl.ds(PAD + c, L)]
                    gP1 = gP[pl.ds(c, L)]
                    sy1 = sxC1 + sxP1
                    sz1 = sxP0 + sxP1
                    wsum = sxC0 + sxC1 + sz1
                    gsum = (gC0 + gC1) + (gP0 + gP1)
                    sx1 = wsum - gsum
                    r = 1.0 / wsum
                    xi = iota + (c & (SX - 1))
                    y = c >> 6
                    yf = y.astype(jnp.float32) - 32.0
                    xf = xi.astype(jnp.float32) - 32.0
                    px = (xf - sx1 * r) * s0
                    py = (yf - sy1 * r) * s1
                    pz = (zf - sz1 * r) * s2
                    m = (wsum != 0.0) & (xi > 0) & (y > 0)
                    pxv[pl.ds(c, L)] = jnp.where(m, px, 0.0)
                    pyv[pl.ds(c, L)] = jnp.where(m, py, 0.0)
                    pzv[pl.ds(c, L)] = jnp.where(m, pz, 0.0)
                    return 0

                lax.fori_loop(0, ROW // L, body, 0)
                pltpu.sync_copy(pxv, out_hbm.at[0, b, z])
                pltpu.sync_copy(pyv, out_hbm.at[1, b, z])
                pltpu.sync_copy(pzv, out_hbm.at[2, b, z])

            return 0

        lax.fori_loop(0, SZ // 2, slab, 0)

    return k


def kernel(grid, output_scaling_factors):
    osf_exp = jnp.repeat(output_scaling_factors, L)
    grid3 = grid.reshape(B, SZ, ROW)
    out = _gridding_reverse_sc()(grid3, osf_exp)
    return out.reshape(3, B, N).transpose(1, 2, 0)


# speedup vs baseline: 34.3133x; 1.4027x over previous
import jax
import jax.numpy as jnp
from jax import lax
from jax.experimental import pallas as pl
from jax.experimental.pallas import tpu as pltpu
from jax.experimental.pallas import tpu_sc as plsc

SX = SY = SZ = 64
ROW = SY * SX
B = 16
N = SX * SY * SZ
PAD = 64
NC, NS, L = 2, 16, 16


def _gridding_reverse_sc():
    mesh = plsc.VectorSubcoreMesh(
        core_axis_name="c", subcore_axis_name="s", num_cores=NC, num_subcores=NS
    )

    @pl.kernel(
        out_type=jax.ShapeDtypeStruct((3, B, SZ, ROW), jnp.float32),
        mesh=mesh,
        compiler_params=pltpu.CompilerParams(
            needs_layout_passes=False, use_tc_tiling_on_sc=False
        ),
        scratch_types=[
            pltpu.VMEM((PAD + ROW,), jnp.float32),
            pltpu.VMEM((PAD + ROW,), jnp.float32),
            pltpu.VMEM((PAD + ROW,), jnp.float32),
            pltpu.VMEM((PAD + ROW,), jnp.float32),
            pltpu.VMEM((ROW,), jnp.float32),
            pltpu.VMEM((ROW,), jnp.float32),
            pltpu.VMEM((ROW,), jnp.float32),
            pltpu.VMEM((3 * L,), jnp.float32),
        ],
    )
    def k(grid_hbm, osf_hbm, out_hbm, gP, gC, sxP, sxC, pxv, pyv, pzv, osf_v):
        wid = lax.axis_index("s") * NC + lax.axis_index("c")
        b = wid >> 1
        z0 = (wid & 1) * (SZ // 2)

        pltpu.sync_copy(osf_hbm, osf_v)
        s0 = osf_v[pl.ds(0, L)]
        s1 = osf_v[pl.ds(L, L)]
        s2 = osf_v[pl.ds(2 * L, L)]

        iota = lax.iota(jnp.int32, L)
        zeros = jnp.zeros((L,), jnp.float32)

        @plsc.parallel_loop(0, ROW, L)
        def _(c):
            pxv[pl.ds(c, L)] = zeros
            pyv[pl.ds(c, L)] = zeros
            pzv[pl.ds(c, L)] = zeros

        def pass1(gbuf, sxbuf):
            @plsc.parallel_loop(0, ROW, L, unroll=4)
            def _(c):
                g0 = gbuf[pl.ds(PAD + c, L)]
                gm = plsc.load_gather(gbuf, [iota + (PAD - 1 + c)])
                sxbuf[pl.ds(PAD + c, L)] = g0 + gm

        def slab(i, _):
            z = z0 + i

            @pl.when(z == 0)
            def _():
                pltpu.sync_copy(pxv, out_hbm.at[0, b, 0])
                pltpu.sync_copy(pyv, out_hbm.at[1, b, 0])
                pltpu.sync_copy(pzv, out_hbm.at[2, b, 0])

            @pl.when(z > 0)
            def _():
                pltpu.sync_copy(grid_hbm.at[b, z - 1], gP.at[pl.ds(PAD, ROW)])
                pltpu.sync_copy(grid_hbm.at[b, z], gC.at[pl.ds(PAD, ROW)])
                pass1(gP, sxP)
                pass1(gC, sxC)
                zf = z.astype(jnp.float32) - 32.0

                @plsc.parallel_loop(0, ROW, L, unroll=4)
                def body(c):
                    sxC0 = sxC[pl.ds(PAD + c, L)]
                    sxC1 = sxC[pl.ds(c, L)]
                    sxP0 = sxP[pl.ds(PAD + c, L)]
                    sxP1 = sxP[pl.ds(c, L)]
                    gC0 = gC[pl.ds(PAD + c, L)]
                    gC1 = gC[pl.ds(c, L)]
                    gP0 = gP[pl.ds(PAD + c, L)]
                    gP1 = gP[pl.ds(c, L)]
                    sy1 = sxC1 + sxP1
                    sz1 = sxP0 + sxP1
                    wsum = sxC0 + sxC1 + sz1
                    gsum = (gC0 + gC1) + (gP0 + gP1)
                    sx1 = wsum - gsum
                    r = 1.0 / wsum
                    xi = iota + (c & (SX - 1))
                    y = c >> 6
                    yf = y.astype(jnp.float32) - 32.0
                    xf = xi.astype(jnp.float32) - 32.0
                    px = (xf - sx1 * r) * s0
                    py = (yf - sy1 * r) * s1
                    pz = (zf - sz1 * r) * s2
                    m = (wsum != 0.0) & (xi > 0) & (y > 0)
                    pxv[pl.ds(c, L)] = jnp.where(m, px, 0.0)
                    pyv[pl.ds(c, L)] = jnp.where(m, py, 0.0)
                    pzv[pl.ds(c, L)] = jnp.where(m, pz, 0.0)

                pltpu.sync_copy(pxv, out_hbm.at[0, b, z])
                pltpu.sync_copy(pyv, out_hbm.at[1, b, z])
                pltpu.sync_copy(pzv, out_hbm.at[2, b, z])

            return 0

        lax.fori_loop(0, SZ // 2, slab, 0)

    return k


def kernel(grid, output_scaling_factors):
    osf_exp = jnp.repeat(output_scaling_factors, L)
    grid3 = grid.reshape(B, SZ, ROW)
    out = _gridding_reverse_sc()(grid3, osf_exp)
    return out.reshape(3, B, N).transpose(1, 2, 0)


# speedup vs baseline: 44.6066x; 1.3000x over previous
import jax
import jax.numpy as jnp
from jax import lax
from jax.experimental import pallas as pl
from jax.experimental.pallas import tpu as pltpu
from jax.experimental.pallas import tpu_sc as plsc

SX = SY = SZ = 64
ROW = SY * SX
B = 16
N = SX * SY * SZ
PAD = 64
NC, NS, L = 2, 16, 16
NPAIR = 16


def _gridding_reverse_sc():
    mesh = plsc.VectorSubcoreMesh(
        core_axis_name="c", subcore_axis_name="s", num_cores=NC, num_subcores=NS
    )

    @pl.kernel(
        out_type=jax.ShapeDtypeStruct((3, B, SZ, ROW), jnp.float32),
        mesh=mesh,
        compiler_params=pltpu.CompilerParams(
            needs_layout_passes=False, use_tc_tiling_on_sc=False
        ),
        scratch_types=[
            pltpu.VMEM((PAD + ROW,), jnp.float32),
            pltpu.VMEM((PAD + ROW,), jnp.float32),
            pltpu.VMEM((PAD + ROW,), jnp.float32),
            pltpu.VMEM((PAD + ROW,), jnp.float32),
            pltpu.VMEM((ROW,), jnp.float32),
            pltpu.VMEM((ROW,), jnp.float32),
            pltpu.VMEM((ROW,), jnp.float32),
            pltpu.VMEM((ROW,), jnp.float32),
            pltpu.VMEM((ROW,), jnp.float32),
            pltpu.VMEM((ROW,), jnp.float32),
            pltpu.VMEM((3 * L,), jnp.float32),
            pltpu.SemaphoreType.DMA,
            pltpu.SemaphoreType.DMA,
            pltpu.SemaphoreType.DMA,
        ],
    )
    def k(grid_hbm, osf_hbm, out_hbm, g0, g1, sx0, sx1,
          pxA, pyA, pzA, pxB, pyB, pzB, osf_v, semIn, semA, semB):
        wid = lax.axis_index("s") * NC + lax.axis_index("c")
        b = wid >> 1
        odd = wid & 1
        zstart = 1 + odd * (SZ // 2)

        pltpu.sync_copy(osf_hbm, osf_v)
        s0 = osf_v[pl.ds(0, L)]
        s1 = osf_v[pl.ds(L, L)]
        s2 = osf_v[pl.ds(2 * L, L)]

        iota = lax.iota(jnp.int32, L)
        zeros = jnp.zeros((L,), jnp.float32)

        @plsc.parallel_loop(0, ROW, L)
        def _(c):
            pxA[pl.ds(c, L)] = zeros
            pyA[pl.ds(c, L)] = zeros
            pzA[pl.ds(c, L)] = zeros

        @pl.when(odd == 1)
        def _():
            pltpu.sync_copy(pxA, out_hbm.at[0, b, 0])
            pltpu.sync_copy(pyA, out_hbm.at[1, b, 0])
            pltpu.sync_copy(pzA, out_hbm.at[2, b, 0])

        def pass1(gbuf, sxbuf):
            @plsc.parallel_loop(0, ROW, L, unroll=4)
            def _(c):
                g0v = gbuf[pl.ds(PAD + c, L)]
                gm = plsc.load_gather(gbuf, [iota + (PAD - 1 + c)])
                sxbuf[pl.ds(PAD + c, L)] = g0v + gm

        def pass2(gP, sxP, gC, sxC, pxv, pyv, pzv, z):
            zf = z.astype(jnp.float32) - 32.0

            @plsc.parallel_loop(0, ROW, L, unroll=4)
            def _(c):
                sxC0 = sxC[pl.ds(PAD + c, L)]
                sxC1 = sxC[pl.ds(c, L)]
                sxP0 = sxP[pl.ds(PAD + c, L)]
                sxP1 = sxP[pl.ds(c, L)]
                gC0 = gC[pl.ds(PAD + c, L)]
                gC1 = gC[pl.ds(c, L)]
                gP0 = gP[pl.ds(PAD + c, L)]
                gP1 = gP[pl.ds(c, L)]
                sy1 = sxC1 + sxP1
                sz1 = sxP0 + sxP1
                wsum = sxC0 + sxC1 + sz1
                gsum = (gC0 + gC1) + (gP0 + gP1)
                sx1v = wsum - gsum
                r = 1.0 / wsum
                xi = iota + (c & (SX - 1))
                y = c >> 6
                yf = y.astype(jnp.float32) - 32.0
                xf = xi.astype(jnp.float32) - 32.0
                px = (xf - sx1v * r) * s0
                py = (yf - sy1 * r) * s1
                pz = (zf - sz1 * r) * s2
                m = (wsum != 0.0) & (xi > 0) & (y > 0)
                pxv[pl.ds(c, L)] = jnp.where(m, px, 0.0)
                pyv[pl.ds(c, L)] = jnp.where(m, py, 0.0)
                pzv[pl.ds(c, L)] = jnp.where(m, pz, 0.0)

        def out_start(pxv, pyv, pzv, z, sem):
            pltpu.async_copy(pxv, out_hbm.at[0, b, z], sem)
            pltpu.async_copy(pyv, out_hbm.at[1, b, z], sem)
            pltpu.async_copy(pzv, out_hbm.at[2, b, z], sem)

        def out_drain(pxv, pyv, pzv, sem):
            pltpu.make_async_copy(pxv, out_hbm.at[0, b, 0], sem).wait()
            pltpu.make_async_copy(pyv, out_hbm.at[1, b, 0], sem).wait()
            pltpu.make_async_copy(pzv, out_hbm.at[2, b, 0], sem).wait()

        pltpu.sync_copy(grid_hbm.at[b, zstart - 1], g0.at[pl.ds(PAD, ROW)])
        pass1(g0, sx0)
        pltpu.async_copy(grid_hbm.at[b, zstart], g1.at[pl.ds(PAD, ROW)], semIn)

        def pair(i, _):
            zA = zstart + 2 * i
            zB = zA + 1

            pltpu.make_async_copy(
                grid_hbm.at[b, zA], g1.at[pl.ds(PAD, ROW)], semIn).wait()
            pass1(g1, sx1)

            @pl.when(i > 0)
            def _():
                out_drain(pxA, pyA, pzA, semA)

            pass2(g0, sx0, g1, sx1, pxA, pyA, pzA, zA)
            out_start(pxA, pyA, pzA, zA, semA)

            @pl.when(zB < SZ)
            def _():
                pltpu.async_copy(grid_hbm.at[b, zB], g0.at[pl.ds(PAD, ROW)], semIn)
                pltpu.make_async_copy(
                    grid_hbm.at[b, zB], g0.at[pl.ds(PAD, ROW)], semIn).wait()
                pass1(g0, sx0)

                @pl.when(i > 0)
                def _():
                    out_drain(pxB, pyB, pzB, semB)

                pass2(g1, sx1, g0, sx0, pxB, pyB, pzB, zB)
                out_start(pxB, pyB, pzB, zB, semB)

                @pl.when(i < NPAIR - 1)
                def _():
                    pltpu.async_copy(
                        grid_hbm.at[b, zA + 2], g1.at[pl.ds(PAD, ROW)], semIn)

            return 0

        lax.fori_loop(0, NPAIR, pair, 0)
        out_drain(pxA, pyA, pzA, semA)
        out_drain(pxB, pyB, pzB, semB)

    return k


def kernel(grid, output_scaling_factors):
    osf_exp = jnp.repeat(output_scaling_factors, L)
    grid3 = grid.reshape(B, SZ, ROW)
    out = _gridding_reverse_sc()(grid3, osf_exp)
    return out.reshape(3, B, N).transpose(1, 2, 0)


# speedup vs baseline: 76.1699x; 1.7076x over previous
import jax
import jax.numpy as jnp
from jax import lax
from jax.experimental import pallas as pl
from jax.experimental.pallas import tpu as pltpu
from jax.experimental.pallas import tpu_sc as plsc

SX = SY = SZ = 64
ROW = SY * SX
B = 16
N = SX * SY * SZ
NC, NS, L = 2, 16, 16
NPAIR = 16
TR, TC_ = B // 8, N // 128
ZC = ROW // 128


def _gridding_reverse_sc():
    mesh = plsc.VectorSubcoreMesh(
        core_axis_name="c", subcore_axis_name="s", num_cores=NC, num_subcores=NS
    )

    @pl.kernel(
        out_type=jax.ShapeDtypeStruct((3, TR, TC_, 8, 128), jnp.float32),
        mesh=mesh,
        compiler_params=pltpu.CompilerParams(
            needs_layout_passes=False, use_tc_tiling_on_sc=False
        ),
        scratch_types=[
            pltpu.VMEM((ZC, 128), jnp.float32),
            pltpu.VMEM((ZC, 128), jnp.float32),
            pltpu.VMEM((ZC, 128), jnp.float32),
            pltpu.VMEM((ZC, 128), jnp.float32),
            pltpu.VMEM((ZC, 128), jnp.float32),
            pltpu.VMEM((ZC, 128), jnp.float32),
            pltpu.VMEM((ZC, 128), jnp.float32),
            pltpu.VMEM((ZC, 128), jnp.float32),
            pltpu.VMEM((ZC, 128), jnp.float32),
            pltpu.VMEM((ZC, 128), jnp.float32),
            pltpu.VMEM((3 * L,), jnp.float32),
            pltpu.SemaphoreType.DMA,
            pltpu.SemaphoreType.DMA,
            pltpu.SemaphoreType.DMA,
        ],
    )
    def k(grid_hbm, osf_hbm, out_hbm, g0, g1, sx0, sx1,
          pxA, pyA, pzA, pxB, pyB, pzB, osf_v, semIn, semA, semB):
        wid = lax.axis_index("s") * NC + lax.axis_index("c")
        b = wid >> 1
        odd = wid & 1
        zstart = 1 + odd * (SZ // 2)
        tr = b >> 3
        sl = b & 7

        pltpu.sync_copy(osf_hbm, osf_v)
        s0 = osf_v[pl.ds(0, L)]
        s1 = osf_v[pl.ds(L, L)]
        s2 = osf_v[pl.ds(2 * L, L)]

        iota = lax.iota(jnp.int32, L)
        zeros = jnp.zeros((L,), jnp.float32)

        def in_slice(z):
            return grid_hbm.at[tr, pl.ds(z * ZC, ZC), sl]

        def out_slice(cc, z):
            return out_hbm.at[cc, tr, pl.ds(z * ZC, ZC), sl]

        @plsc.parallel_loop(0, ROW, L)
        def _(c):
            pxA[c >> 7, pl.ds(c & 127, L)] = zeros
            pyA[c >> 7, pl.ds(c & 127, L)] = zeros
            pzA[c >> 7, pl.ds(c & 127, L)] = zeros

        @pl.when(odd == 1)
        def _():
            pltpu.sync_copy(pxA, out_slice(0, 0))
            pltpu.sync_copy(pyA, out_slice(1, 0))
            pltpu.sync_copy(pzA, out_slice(2, 0))

        def pass1(gbuf, sxbuf):
            @plsc.parallel_loop(0, ROW, L, unroll=4)
            def _(c):
                g0v = gbuf[c >> 7, pl.ds(c & 127, L)]
                f = jnp.maximum(iota + (c - 1), 0)
                gm = plsc.load_gather(gbuf, [f >> 7, f & 127])
                sxbuf[c >> 7, pl.ds(c & 127, L)] = g0v + gm

        def pass2(gP, sxP, gC, sxC, pxv, pyv, pzv, z):
            zf = z.astype(jnp.float32) - 32.0

            for kk in range(4):
                pxv[0, pl.ds(kk * L, L)] = zeros
                pyv[0, pl.ds(kk * L, L)] = zeros
                pzv[0, pl.ds(kk * L, L)] = zeros

            @plsc.parallel_loop(64, ROW, L, unroll=4)
            def _(c):
                r0 = c >> 7
                c0 = c & 127
                d = c - 64
                r1 = d >> 7
                c1 = d & 127
                sxC0 = sxC[r0, pl.ds(c0, L)]
                sxC1 = sxC[r1, pl.ds(c1, L)]
                sxP0 = sxP[r0, pl.ds(c0, L)]
                sxP1 = sxP[r1, pl.ds(c1, L)]
                gC0 = gC[r0, pl.ds(c0, L)]
                gC1 = gC[r1, pl.ds(c1, L)]
                gP0 = gP[r0, pl.ds(c0, L)]
                gP1 = gP[r1, pl.ds(c1, L)]
                sy1 = sxC1 + sxP1
                sz1 = sxP0 + sxP1
                wsum = sxC0 + sxC1 + sz1
                gsum = (gC0 + gC1) + (gP0 + gP1)
                sx1v = wsum - gsum
                r = 1.0 / wsum
                xi = iota + (c & (SX - 1))
                y = c >> 6
                yf = y.astype(jnp.float32) - 32.0
                xf = xi.astype(jnp.float32) - 32.0
                px = (xf - sx1v * r) * s0
                py = (yf - sy1 * r) * s1
                pz = (zf - sz1 * r) * s2
                m = (wsum != 0.0) & (xi > 0)
                pxv[r0, pl.ds(c0, L)] = jnp.where(m, px, 0.0)
                pyv[r0, pl.ds(c0, L)] = jnp.where(m, py, 0.0)
                pzv[r0, pl.ds(c0, L)] = jnp.where(m, pz, 0.0)

        def out_start(pxv, pyv, pzv, z, sem):
            pltpu.async_copy(pxv, out_slice(0, z), sem)
            pltpu.async_copy(pyv, out_slice(1, z), sem)
            pltpu.async_copy(pzv, out_slice(2, z), sem)

        def out_drain(pxv, pyv, pzv, sem):
            pltpu.make_async_copy(pxv, out_slice(0, 0), sem).wait()
            pltpu.make_async_copy(pyv, out_slice(1, 0), sem).wait()
            pltpu.make_async_copy(pzv, out_slice(2, 0), sem).wait()

        pltpu.sync_copy(in_slice(zstart - 1), g0)
        pass1(g0, sx0)
        pltpu.async_copy(in_slice(zstart), g1, semIn)

        def pair(i, _):
            zA = zstart + 2 * i
            zB = zA + 1

            pltpu.make_async_copy(in_slice(zA), g1, semIn).wait()
            pass1(g1, sx1)

            @pl.when(i > 0)
            def _():
                out_drain(pxA, pyA, pzA, semA)

            pass2(g0, sx0, g1, sx1, pxA, pyA, pzA, zA)
            out_start(pxA, pyA, pzA, zA, semA)

            @pl.when(zB < SZ)
            def _():
                pltpu.async_copy(in_slice(zB), g0, semIn)
                pltpu.make_async_copy(in_slice(zB), g0, semIn).wait()
                pass1(g0, sx0)

                @pl.when(i > 0)
                def _():
                    out_drain(pxB, pyB, pzB, semB)

                pass2(g1, sx1, g0, sx0, pxB, pyB, pzB, zB)
                out_start(pxB, pyB, pzB, zB, semB)

                @pl.when(i < NPAIR - 1)
                def _():
                    pltpu.async_copy(in_slice(zA + 2), g1, semIn)

            return 0

        lax.fori_loop(0, NPAIR, pair, 0)
        out_drain(pxA, pyA, pzA, semA)
        out_drain(pxB, pyB, pzB, semB)

    return k


def kernel(grid, output_scaling_factors):
    osf_exp = jnp.repeat(output_scaling_factors, L)
    grid4 = grid.reshape(TR, 8, TC_, 128).transpose(0, 2, 1, 3)
    out5 = _gridding_reverse_sc()(grid4, osf_exp)
    out = out5.transpose(0, 1, 3, 2, 4).reshape(3, B, N)
    return out.transpose(1, 2, 0)


# speedup vs baseline: 76.9593x; 1.0104x over previous
import jax
import jax.numpy as jnp
from jax import lax
from jax.experimental import pallas as pl
from jax.experimental.pallas import tpu as pltpu
from jax.experimental.pallas import tpu_sc as plsc

SX = SY = SZ = 64
ROW = SY * SX
B = 16
N = SX * SY * SZ
NC, NS, L = 2, 16, 16
NPAIR = 16
TR, TC_ = B // 8, N // 128
ZC = ROW // 128


def _gridding_reverse_sc():
    mesh = plsc.VectorSubcoreMesh(
        core_axis_name="c", subcore_axis_name="s", num_cores=NC, num_subcores=NS
    )

    @pl.kernel(
        out_type=jax.ShapeDtypeStruct((3, TR, TC_, 8, 128), jnp.float32),
        mesh=mesh,
        compiler_params=pltpu.CompilerParams(
            needs_layout_passes=False, use_tc_tiling_on_sc=False
        ),
        scratch_types=[
            pltpu.VMEM((ZC, 128), jnp.float32),
            pltpu.VMEM((ZC, 128), jnp.float32),
            pltpu.VMEM((ZC, 128), jnp.float32),
            pltpu.VMEM((ZC, 128), jnp.float32),
            pltpu.VMEM((ZC, 128), jnp.float32),
            pltpu.VMEM((ZC, 128), jnp.float32),
            pltpu.VMEM((ZC, 128), jnp.float32),
            pltpu.VMEM((ZC, 128), jnp.float32),
            pltpu.VMEM((ZC, 128), jnp.float32),
            pltpu.VMEM((ZC, 128), jnp.float32),
            pltpu.VMEM((3 * L,), jnp.float32),
            pltpu.SemaphoreType.DMA,
            pltpu.SemaphoreType.DMA,
            pltpu.SemaphoreType.DMA,
        ],
    )
    def k(grid_hbm, osf_hbm, out_hbm, g0, g1, sx0, sx1,
          pxA, pyA, pzA, pxB, pyB, pzB, osf_v, semIn, semA, semB):
        wid = lax.axis_index("s") * NC + lax.axis_index("c")
        b = wid >> 1
        odd = wid & 1
        zstart = 1 + odd * (SZ // 2)
        tr = b >> 3
        sl = b & 7

        pltpu.sync_copy(osf_hbm, osf_v)
        s0 = osf_v[pl.ds(0, L)]
        s1 = osf_v[pl.ds(L, L)]
        s2 = osf_v[pl.ds(2 * L, L)]

        iota = lax.iota(jnp.int32, L)
        zeros = jnp.zeros((L,), jnp.float32)

        def in_slice(z):
            return grid_hbm.at[tr, pl.ds(z * ZC, ZC), sl]

        def out_slice(cc, z):
            return out_hbm.at[cc, tr, pl.ds(z * ZC, ZC), sl]

        @plsc.parallel_loop(0, ROW, L)
        def _(c):
            pxA[c >> 7, pl.ds(c & 127, L)] = zeros
            pyA[c >> 7, pl.ds(c & 127, L)] = zeros
            pzA[c >> 7, pl.ds(c & 127, L)] = zeros

        @pl.when(odd == 1)
        def _():
            pltpu.sync_copy(pxA, out_slice(0, 0))
            pltpu.sync_copy(pyA, out_slice(1, 0))
            pltpu.sync_copy(pzA, out_slice(2, 0))

        def pass1(gbuf, sxbuf):
            @plsc.parallel_loop(0, ROW, L, unroll=8)
            def _(c):
                g0v = gbuf[c >> 7, pl.ds(c & 127, L)]
                f = jnp.maximum(iota + (c - 1), 0)
                gm = plsc.load_gather(gbuf, [f >> 7, f & 127])
                sxbuf[c >> 7, pl.ds(c & 127, L)] = g0v + gm

        def pass2(gP, sxP, gC, sxC, pxv, pyv, pzv, z):
            zf = z.astype(jnp.float32) - 32.0

            for kk in range(4):
                pxv[0, pl.ds(kk * L, L)] = zeros
                pyv[0, pl.ds(kk * L, L)] = zeros
                pzv[0, pl.ds(kk * L, L)] = zeros

            @plsc.parallel_loop(64, ROW, L, unroll=4)
            def _(c):
                r0 = c >> 7
                c0 = c & 127
                d = c - 64
                r1 = d >> 7
                c1 = d & 127
                sxC0 = sxC[r0, pl.ds(c0, L)]
                sxC1 = sxC[r1, pl.ds(c1, L)]
                sxP0 = sxP[r0, pl.ds(c0, L)]
                sxP1 = sxP[r1, pl.ds(c1, L)]
                gC0 = gC[r0, pl.ds(c0, L)]
                gC1 = gC[r1, pl.ds(c1, L)]
                gP0 = gP[r0, pl.ds(c0, L)]
                gP1 = gP[r1, pl.ds(c1, L)]
                sy1 = sxC1 + sxP1
                sz1 = sxP0 + sxP1
                wsum = sxC0 + sxC1 + sz1
                gsum = (gC0 + gC1) + (gP0 + gP1)
                sx1v = wsum - gsum
                r = 1.0 / wsum
                xi = iota + (c & (SX - 1))
                y = c >> 6
                yf = y.astype(jnp.float32) - 32.0
                xf = xi.astype(jnp.float32) - 32.0
                px = (xf - sx1v * r) * s0
                py = (yf - sy1 * r) * s1
                pz = (zf - sz1 * r) * s2
                m = (wsum != 0.0) & (xi > 0)
                pxv[r0, pl.ds(c0, L)] = jnp.where(m, px, 0.0)
                pyv[r0, pl.ds(c0, L)] = jnp.where(m, py, 0.0)
                pzv[r0, pl.ds(c0, L)] = jnp.where(m, pz, 0.0)

        def out_start(pxv, pyv, pzv, z, sem):
            pltpu.async_copy(pxv, out_slice(0, z), sem)
            pltpu.async_copy(pyv, out_slice(1, z), sem)
            pltpu.async_copy(pzv, out_slice(2, z), sem)

        def out_drain(pxv, pyv, pzv, sem):
            pltpu.make_async_copy(pxv, out_slice(0, 0), sem).wait()
            pltpu.make_async_copy(pyv, out_slice(1, 0), sem).wait()
            pltpu.make_async_copy(pzv, out_slice(2, 0), sem).wait()

        pltpu.sync_copy(in_slice(zstart - 1), g0)
        pass1(g0, sx0)
        pltpu.async_copy(in_slice(zstart), g1, semIn)

        def pair(i, _):
            zA = zstart + 2 * i
            zB = zA + 1

            pltpu.make_async_copy(in_slice(zA), g1, semIn).wait()
            pass1(g1, sx1)

            @pl.when(i > 0)
            def _():
                out_drain(pxA, pyA, pzA, semA)

            pass2(g0, sx0, g1, sx1, pxA, pyA, pzA, zA)
            out_start(pxA, pyA, pzA, zA, semA)

            @pl.when(zB < SZ)
            def _():
                pltpu.async_copy(in_slice(zB), g0, semIn)
                pltpu.make_async_copy(in_slice(zB), g0, semIn).wait()
                pass1(g0, sx0)

                @pl.when(i > 0)
                def _():
                    out_drain(pxB, pyB, pzB, semB)

                pass2(g1, sx1, g0, sx0, pxB, pyB, pzB, zB)
                out_start(pxB, pyB, pzB, zB, semB)

                @pl.when(i < NPAIR - 1)
                def _():
                    pltpu.async_copy(in_slice(zA + 2), g1, semIn)

            return 0

        lax.fori_loop(0, NPAIR, pair, 0)
        out_drain(pxA, pyA, pzA, semA)
        out_drain(pxB, pyB, pzB, semB)

    return k


def kernel(grid, output_scaling_factors):
    osf_exp = jnp.repeat(output_scaling_factors, L)
    grid4 = grid.reshape(TR, 8, TC_, 128).transpose(0, 2, 1, 3)
    out5 = _gridding_reverse_sc()(grid4, osf_exp)
    out = out5.transpose(0, 1, 3, 2, 4).reshape(3, B, N)
    return out.transpose(1, 2, 0)
